# baseline (speedup 1.0000x reference)
import functools

import jax
import jax.numpy as jnp
from jax import lax
from jax.experimental import pallas as pl
from jax.experimental.pallas import tpu as pltpu
from jax.experimental.pallas import tpu_sc as plsc

_N = 8192
_M = 8192
_D = 256
_RB = 256
_G = _N // _RB
_MH = _M // 2
_TW = _N // 16
_LC = _TW // 2


def _tc_body(qt_ref, keys_ref, p_ref, concat_ref, qr_ref,
             t1i_ref, t2i_ref, t1v_ref, t2v_ref, qn2_ref, qsum_ref,
             colmax_ref, kn2_ref, ksum_ref):
    i = pl.program_id(0)
    qraw = qt_ref[...]
    n2 = jnp.sum(qraw * qraw, axis=1, keepdims=True)
    norm = jnp.maximum(jnp.sqrt(n2), 1e-12)
    q = qraw / norm
    qr_ref[...] = q
    keys = keys_ref[...]
    s = lax.dot_general(q, keys, (((1,), (1,)), ((), ())),
                        preferred_element_type=jnp.float32)
    rowmax = jnp.max(s, axis=1, keepdims=True)
    e = jnp.exp(s - rowmax)
    ones_m = jnp.ones((_M, 1), jnp.float32)
    rowsum = lax.dot_general(e, ones_m, (((1,), (0,)), ((), ())),
                             preferred_element_type=jnp.float32)
    p = e * (1.0 / rowsum)
    p_ref[...] = p
    concat_ref[...] = lax.dot_general(p, keys, (((1,), (0,)), ((), ())),
                                      preferred_element_type=jnp.float32)
    col_ids = lax.broadcasted_iota(jnp.int32, s.shape, 1)
    t1 = jnp.min(jnp.where(s == rowmax, col_ids, _M), axis=1)
    masked = jnp.where(col_ids == t1[:, None], -jnp.inf, s)
    t2v = jnp.max(masked, axis=1, keepdims=True)
    t2 = jnp.min(jnp.where(masked == t2v, col_ids, _M), axis=1)
    t1i_ref[0] = t1[:, None]
    t2i_ref[0] = t2[:, None]
    t1v_ref[0] = rowmax
    t2v_ref[0] = t2v
    qn2_ref[0] = jnp.sum(q * q, axis=1, keepdims=True)
    qsum_ref[0] = jnp.sum(q, axis=1, keepdims=True)
    bm = jnp.max(s, axis=0, keepdims=True)

    @pl.when(i == 0)
    def _():
        colmax_ref[...] = bm
        kn2_ref[...] = jnp.sum(keys * keys, axis=1, keepdims=True)
        ksum_ref[...] = jnp.sum(keys, axis=1, keepdims=True)

    @pl.when(i > 0)
    def _():
        colmax_ref[...] = jnp.maximum(colmax_ref[...], bm)


def _tc_call(qt, keys):
    f32 = jnp.float32
    out_shapes = (
        jax.ShapeDtypeStruct((_N, _M), f32),
        jax.ShapeDtypeStruct((_N, _D), f32),
        jax.ShapeDtypeStruct((_N, _D), f32),
        jax.ShapeDtypeStruct((_G, _RB, 1), jnp.int32),
        jax.ShapeDtypeStruct((_G, _RB, 1), jnp.int32),
        jax.ShapeDtypeStruct((_G, _RB, 1), f32),
        jax.ShapeDtypeStruct((_G, _RB, 1), f32),
        jax.ShapeDtypeStruct((_G, _RB, 1), f32),
        jax.ShapeDtypeStruct((_G, _RB, 1), f32),
        jax.ShapeDtypeStruct((1, _M), f32),
        jax.ShapeDtypeStruct((_M, 1), f32),
        jax.ShapeDtypeStruct((_M, 1), f32),
    )
    stat_spec = pl.BlockSpec((1, _RB, 1), lambda i: (i, 0, 0))
    full2 = lambda a, b: pl.BlockSpec((a, b), lambda i: (0, 0))
    return pl.pallas_call(
        _tc_body,
        grid=(_G,),
        compiler_params=pltpu.CompilerParams(
            dimension_semantics=("arbitrary",),
            vmem_limit_bytes=100 * 1024 * 1024,
        ),
        in_specs=[
            pl.BlockSpec((_RB, _D), lambda i: (i, 0)),
            full2(_M, _D),
        ],
        out_specs=[
            pl.BlockSpec((_RB, _M), lambda i: (i, 0)),
            pl.BlockSpec((_RB, _D), lambda i: (i, 0)),
            pl.BlockSpec((_RB, _D), lambda i: (i, 0)),
            stat_spec, stat_spec, stat_spec, stat_spec, stat_spec, stat_spec,
            full2(1, _M), full2(_M, 1), full2(_M, 1),
        ],
        out_shape=out_shapes,
    )(qt, keys)


def _nsqrt(x):
    x = jnp.maximum(x, 0.0)
    i = lax.bitcast_convert_type(x, jnp.int32)
    i = (i >> 1) + 0x1FBD1DF6
    y = lax.bitcast_convert_type(i, jnp.float32)
    for _ in range(4):
        y = 0.5 * (y + x / y)
    return y


_B = 8
_SPT = _M // 32
_LIST = _N + 32


def _sc_body(qr_hbm, t1i_hbm, t2i_hbm, t1v_hbm, t2v_hbm, qn2_hbm, qsum_hbm,
             colmax_hbm, kn2_hbm, ksum_hbm,
             upd_hbm, lossp_hbm,
             accum, qbuf, colmax_v, kn2_v, ksum_v,
             t1i_v, t1v_v, t2i_v, t2v_v, qn2_v, qsum_v,
             toklist, lbuf, sem):
    c = lax.axis_index("c")
    s = lax.axis_index("s")
    wid = c * 16 + s
    slot0 = wid * _SPT
    loff = wid * _LC
    pltpu.sync_copy(colmax_hbm, colmax_v)
    pltpu.sync_copy(kn2_hbm, kn2_v)
    pltpu.sync_copy(ksum_hbm, ksum_v)
    pltpu.sync_copy(t1i_hbm, t1i_v)
    pltpu.sync_copy(t1v_hbm, t1v_v)
    pltpu.sync_copy(t2i_hbm.at[pl.ds(loff, _LC)], t2i_v)
    pltpu.sync_copy(t2v_hbm.at[pl.ds(loff, _LC)], t2v_v)
    pltpu.sync_copy(qn2_hbm.at[pl.ds(loff, _LC)], qn2_v)
    pltpu.sync_copy(qsum_hbm.at[pl.ds(loff, _LC)], qsum_v)

    zero16 = jnp.zeros((16,), jnp.float32)
    lanes = lax.iota(jnp.int32, 16)

    def zloop(k, _):
        for j in range(_D // 16):
            accum[k, pl.ds(j * 16, 16)] = zero16
        return 0
    lax.fori_loop(0, _SPT, zloop, 0)

    def scan(k, off):
        gi = t1i_v[pl.ds(k * 16, 16)]
        lg = gi - slot0
        m = (lg >= 0) & (lg < _SPT)
        plsc.store_compressed(toklist.at[pl.ds(off, 16)],
                              k * 16 + lanes, mask=m)
        cnt = plsc.all_reduce_population_count(m)
        return off + cnt[0]
    count = lax.fori_loop(0, _N // 16, scan, jnp.int32(0))
    toklist[pl.ds(count, 16)] = jnp.zeros((16,), jnp.int32)

    nchunks = (count + 15) // 16

    def proc(k, _):
        ids = toklist[pl.ds(k * 16, 16)]
        gi = plsc.load_gather(t1i_v, [ids])
        t1v16 = plsc.load_gather(t1v_v, [ids])
        cm = plsc.load_gather(colmax_v, [gi])
        wgt = jnp.exp(t1v16 - cm)
        rem = count - k * 16
        wgt = jnp.where(lanes < rem, wgt, 0.0)
        gl = jnp.clip(gi - slot0, 0, _SPT - 1)
        pltpu.async_copy(qr_hbm.at[ids], qbuf, sem).wait()
        for i in range(16):
            wi = wgt[i]
            gli = gl[i]
            for j in range(_D // 16):
                sl = pl.ds(j * 16, 16)
                plsc.addupdate(accum.at[gli, sl], qbuf[i, sl] * wi)
        return 0
    lax.fori_loop(0, nchunks, proc, 0)

    wcps = [
        pltpu.async_copy(accum, upd_hbm.at[b, pl.ds(slot0, _SPT)], sem)
        for b in range(_B)
    ]
    for cp in wcps:
        cp.wait()

    def lchunk(k, carry):
        a1, a2 = carry
        base = k * 16
        gi = t1i_v[pl.ds(loff + base, 16)]
        t1v16 = t1v_v[pl.ds(loff + base, 16)]
        g2i = t2i_v[pl.ds(base, 16)]
        t2v16 = t2v_v[pl.ds(base, 16)]
        qn216 = qn2_v[pl.ds(base, 16)]
        qs16 = qsum_v[pl.ds(base, 16)]
        kn2g = plsc.load_gather(kn2_v, [gi])
        kn2g2 = plsc.load_gather(kn2_v, [g2i])
        ksg = plsc.load_gather(ksum_v, [gi])
        ksg2 = plsc.load_gather(ksum_v, [g2i])
        l1 = qn216 - 2.0 * t1v16 + kn2g
        dp2 = l1 + 2e-6 * (qs16 - ksg) + (_D * 1e-12)
        dn2 = (qn216 - 2.0 * t2v16 + kn2g2) + 2e-6 * (qs16 - ksg2) + (_D * 1e-12)
        tl = jnp.maximum(_nsqrt(dp2) - _nsqrt(dn2) + 1.0, 0.0)
        return (a1 + l1, a2 + tl)

    zv = jnp.zeros((16,), jnp.float32)
    a1, a2 = lax.fori_loop(0, _LC // 16, lchunk, (zv, zv))
    s1 = jnp.sum(a1, axis=0)
    s2 = jnp.sum(a2, axis=0)
    lbuf[...] = jnp.where(lanes == 0, s1, jnp.where(lanes == 1, s2, 0.0))
    pltpu.sync_copy(lbuf, lossp_hbm.at[wid])


def _sc_call(qr, t1i, t2i, t1v, t2v, qn2, qsum, colmax, kn2, ksum):
    f32 = jnp.float32
    mesh = plsc.VectorSubcoreMesh(core_axis_name="c", subcore_axis_name="s")
    fn = functools.partial(
        pl.kernel,
        mesh=mesh,
        compiler_params=pltpu.CompilerParams(needs_layout_passes=False),
        out_type=[
            jax.ShapeDtypeStruct((_B, _M, _D), f32),
            jax.ShapeDtypeStruct((32, 16), f32),
        ],
        scratch_types=[
            pltpu.VMEM((_SPT, _D), f32),
            pltpu.VMEM((16, _D), f32),
            pltpu.VMEM((_M,), f32),
            pltpu.VMEM((_M,), f32),
            pltpu.VMEM((_M,), f32),
            pltpu.VMEM((_N,), jnp.int32),
            pltpu.VMEM((_N,), f32),
            pltpu.VMEM((_LC,), jnp.int32),
            pltpu.VMEM((_LC,), f32),
            pltpu.VMEM((_LC,), f32),
            pltpu.VMEM((_LC,), f32),
            pltpu.VMEM((_LIST,), jnp.int32),
            pltpu.VMEM((16,), f32),
            pltpu.SemaphoreType.DMA,
        ],
    )(_sc_body)
    return fn(qr, t1i, t2i, t1v, t2v, qn2, qsum, colmax, kn2, ksum)


def kernel(query, keys_b):
    keys = keys_b[0]
    B, d, h, w = query.shape
    qt = jnp.transpose(query, (0, 2, 3, 1)).reshape(B * h * w, d)
    (p, concat, qr, t1i, t2i, t1v, t2v, qn2, qsum,
     colmax, kn2, ksum) = _tc_call(qt, keys)
    upd, lossp = _sc_call(
        qr,
        t1i.reshape(_N), t2i.reshape(_N),
        t1v.reshape(_N), t2v.reshape(_N),
        qn2.reshape(_N), qsum.reshape(_N),
        colmax.reshape(_M), kn2.reshape(_M), ksum.reshape(_M),
    )
    updated_memory_b = upd
    score_memory_b = p.reshape(B, h, w, _M)
    updated_query = jnp.concatenate([qr, concat], axis=1).reshape(
        B, h, w, 2 * d).transpose(0, 3, 1, 2)
    loss_list = lossp[:, :2].reshape(B, 4, 2).sum(axis=1) / (h * w)
    return updated_query, updated_memory_b, score_memory_b, loss_list


# speedup vs baseline: 1.0387x; 1.0387x over previous
import functools

import jax
import jax.numpy as jnp
from jax import lax
from jax.experimental import pallas as pl
from jax.experimental.pallas import tpu as pltpu
from jax.experimental.pallas import tpu_sc as plsc

_N = 8192
_M = 8192
_D = 256
_RB = 256
_G = _N // _RB
_TW = _N // 16
_LC = _TW // 2


def _tc_body(qt_ref, keys_ref, p_ref, concat_ref, qr_ref,
             t1i_ref, t2i_ref, t1v_ref, t2v_ref, qn2_ref, qsum_ref,
             colmax_ref, kn2_ref, ksum_ref):
    i = pl.program_id(0)
    qraw = qt_ref[...]
    n2 = jnp.sum(qraw * qraw, axis=1, keepdims=True)
    norm = jnp.maximum(jnp.sqrt(n2), 1e-12)
    q = qraw / norm
    qr_ref[...] = q
    keys = keys_ref[...]
    s = lax.dot_general(q, keys, (((1,), (1,)), ((), ())),
                        preferred_element_type=jnp.float32)
    rowmax = jnp.max(s, axis=1, keepdims=True)
    e = jnp.exp(s - rowmax)
    ones_m = jnp.ones((_M, 1), jnp.float32)
    rowsum = lax.dot_general(e, ones_m, (((1,), (0,)), ((), ())),
                             preferred_element_type=jnp.float32)
    p = e * (1.0 / rowsum)
    p_ref[...] = p
    concat_ref[...] = lax.dot_general(p, keys, (((1,), (0,)), ((), ())),
                                      preferred_element_type=jnp.float32)
    col_ids = lax.broadcasted_iota(jnp.int32, s.shape, 1)
    t1 = jnp.min(jnp.where(s == rowmax, col_ids, _M), axis=1)
    masked = jnp.where(col_ids == t1[:, None], -jnp.inf, s)
    t2v = jnp.max(masked, axis=1, keepdims=True)
    t2 = jnp.min(jnp.where(masked == t2v, col_ids, _M), axis=1)
    t1i_ref[0] = t1[:, None]
    t2i_ref[0] = t2[:, None]
    t1v_ref[0] = rowmax
    t2v_ref[0] = t2v
    qn2_ref[0] = jnp.sum(q * q, axis=1, keepdims=True)
    qsum_ref[0] = jnp.sum(q, axis=1, keepdims=True)
    bm = jnp.max(s, axis=0, keepdims=True)

    @pl.when(i == 0)
    def _():
        colmax_ref[...] = bm
        kn2_ref[...] = jnp.sum(keys * keys, axis=1, keepdims=True)
        ksum_ref[...] = jnp.sum(keys, axis=1, keepdims=True)

    @pl.when(i > 0)
    def _():
        colmax_ref[...] = jnp.maximum(colmax_ref[...], bm)


def _tc_call(qt, keys):
    f32 = jnp.float32
    out_shapes = (
        jax.ShapeDtypeStruct((_N, _M), f32),
        jax.ShapeDtypeStruct((_N, _D), f32),
        jax.ShapeDtypeStruct((_N, _D), f32),
        jax.ShapeDtypeStruct((_G, _RB, 1), jnp.int32),
        jax.ShapeDtypeStruct((_G, _RB, 1), jnp.int32),
        jax.ShapeDtypeStruct((_G, _RB, 1), f32),
        jax.ShapeDtypeStruct((_G, _RB, 1), f32),
        jax.ShapeDtypeStruct((_G, _RB, 1), f32),
        jax.ShapeDtypeStruct((_G, _RB, 1), f32),
        jax.ShapeDtypeStruct((1, _M), f32),
        jax.ShapeDtypeStruct((_M, 1), f32),
        jax.ShapeDtypeStruct((_M, 1), f32),
    )
    stat_spec = pl.BlockSpec((1, _RB, 1), lambda i: (i, 0, 0))
    full2 = lambda a, b: pl.BlockSpec((a, b), lambda i: (0, 0))
    return pl.pallas_call(
        _tc_body,
        grid=(_G,),
        compiler_params=pltpu.CompilerParams(
            dimension_semantics=("arbitrary",),
            vmem_limit_bytes=100 * 1024 * 1024,
        ),
        in_specs=[
            pl.BlockSpec((_RB, _D), lambda i: (i, 0)),
            full2(_M, _D),
        ],
        out_specs=[
            pl.BlockSpec((_RB, _M), lambda i: (i, 0)),
            pl.BlockSpec((_RB, _D), lambda i: (i, 0)),
            pl.BlockSpec((_RB, _D), lambda i: (i, 0)),
            stat_spec, stat_spec, stat_spec, stat_spec, stat_spec, stat_spec,
            full2(1, _M), full2(_M, 1), full2(_M, 1),
        ],
        out_shape=out_shapes,
    )(qt, keys)


def _nsqrt(x):
    x = jnp.maximum(x, 0.0)
    i = lax.bitcast_convert_type(x, jnp.int32)
    i = (i >> 1) + 0x1FBD1DF6
    y = lax.bitcast_convert_type(i, jnp.float32)
    for _ in range(4):
        y = 0.5 * (y + x / y)
    return y


_SPT = _M // 32
_LIST = _N + 32


def _sc_body(qr_hbm, t1i_hbm, t2i_hbm, t1v_hbm, t2v_hbm, qn2_hbm, qsum_hbm,
             colmax_hbm, kn2_hbm, ksum_hbm,
             upd_hbm, lossp_hbm,
             accum, qbuf, colmax_v, kn2_v, ksum_v,
             t1i_v, t1v_v, t2i_v, t2v_v, qn2_v, qsum_v,
             toklist, lbuf, sem):
    c = lax.axis_index("c")
    s = lax.axis_index("s")
    wid = c * 16 + s
    slot0 = wid * _SPT
    loff = wid * _LC
    pltpu.sync_copy(colmax_hbm, colmax_v)
    pltpu.sync_copy(kn2_hbm, kn2_v)
    pltpu.sync_copy(ksum_hbm, ksum_v)
    pltpu.sync_copy(t1i_hbm, t1i_v)
    pltpu.sync_copy(t1v_hbm, t1v_v)
    pltpu.sync_copy(t2i_hbm.at[pl.ds(loff, _LC)], t2i_v)
    pltpu.sync_copy(t2v_hbm.at[pl.ds(loff, _LC)], t2v_v)
    pltpu.sync_copy(qn2_hbm.at[pl.ds(loff, _LC)], qn2_v)
    pltpu.sync_copy(qsum_hbm.at[pl.ds(loff, _LC)], qsum_v)

    zero16 = jnp.zeros((16,), jnp.float32)
    lanes = lax.iota(jnp.int32, 16)

    def zloop(k, _):
        for j in range(_D // 16):
            accum[k, pl.ds(j * 16, 16)] = zero16
        return 0
    lax.fori_loop(0, _SPT, zloop, 0)

    def scan(k, off):
        gi = t1i_v[pl.ds(k * 16, 16)]
        lg = gi - slot0
        m = (lg >= 0) & (lg < _SPT)
        plsc.store_compressed(toklist.at[pl.ds(off, 16)],
                              k * 16 + lanes, mask=m)
        cnt = plsc.all_reduce_population_count(m)
        return off + cnt[0]
    count = lax.fori_loop(0, _N // 16, scan, jnp.int32(0))
    toklist[pl.ds(count, 16)] = jnp.zeros((16,), jnp.int32)

    nchunks = (count + 15) // 16

    def proc(k, _):
        ids = toklist[pl.ds(k * 16, 16)]
        gi = plsc.load_gather(t1i_v, [ids])
        t1v16 = plsc.load_gather(t1v_v, [ids])
        cm = plsc.load_gather(colmax_v, [gi])
        wgt = jnp.exp(t1v16 - cm)
        rem = count - k * 16
        wgt = jnp.where(lanes < rem, wgt, 0.0)
        gl = jnp.clip(gi - slot0, 0, _SPT - 1)
        pltpu.async_copy(qr_hbm.at[ids], qbuf, sem).wait()
        for i in range(16):
            wi = wgt[i]
            gli = gl[i]
            for j in range(_D // 16):
                sl = pl.ds(j * 16, 16)
                plsc.addupdate(accum.at[gli, sl], qbuf[i, sl] * wi)
        return 0
    lax.fori_loop(0, nchunks, proc, 0)

    pltpu.sync_copy(accum, upd_hbm.at[pl.ds(slot0, _SPT)])

    def lchunk(k, carry):
        a1, a2 = carry
        base = k * 16
        gi = t1i_v[pl.ds(loff + base, 16)]
        t1v16 = t1v_v[pl.ds(loff + base, 16)]
        g2i = t2i_v[pl.ds(base, 16)]
        t2v16 = t2v_v[pl.ds(base, 16)]
        qn216 = qn2_v[pl.ds(base, 16)]
        qs16 = qsum_v[pl.ds(base, 16)]
        kn2g = plsc.load_gather(kn2_v, [gi])
        kn2g2 = plsc.load_gather(kn2_v, [g2i])
        ksg = plsc.load_gather(ksum_v, [gi])
        ksg2 = plsc.load_gather(ksum_v, [g2i])
        l1 = qn216 - 2.0 * t1v16 + kn2g
        dp2 = l1 + 2e-6 * (qs16 - ksg) + (_D * 1e-12)
        dn2 = (qn216 - 2.0 * t2v16 + kn2g2) + 2e-6 * (qs16 - ksg2) + (_D * 1e-12)
        tl = jnp.maximum(_nsqrt(dp2) - _nsqrt(dn2) + 1.0, 0.0)
        return (a1 + l1, a2 + tl)

    zv = jnp.zeros((16,), jnp.float32)
    a1, a2 = lax.fori_loop(0, _LC // 16, lchunk, (zv, zv))
    s1 = jnp.sum(a1, axis=0)
    s2 = jnp.sum(a2, axis=0)
    lbuf[...] = jnp.where(lanes == 0, s1, jnp.where(lanes == 1, s2, 0.0))
    pltpu.sync_copy(lbuf, lossp_hbm.at[wid])


def _sc_call(qr, t1i, t2i, t1v, t2v, qn2, qsum, colmax, kn2, ksum):
    f32 = jnp.float32
    mesh = plsc.VectorSubcoreMesh(core_axis_name="c", subcore_axis_name="s")
    fn = functools.partial(
        pl.kernel,
        mesh=mesh,
        compiler_params=pltpu.CompilerParams(needs_layout_passes=False),
        out_type=[
            jax.ShapeDtypeStruct((_M, _D), f32),
            jax.ShapeDtypeStruct((32, 16), f32),
        ],
        scratch_types=[
            pltpu.VMEM((_SPT, _D), f32),
            pltpu.VMEM((16, _D), f32),
            pltpu.VMEM((_M,), f32),
            pltpu.VMEM((_M,), f32),
            pltpu.VMEM((_M,), f32),
            pltpu.VMEM((_N,), jnp.int32),
            pltpu.VMEM((_N,), f32),
            pltpu.VMEM((_LC,), jnp.int32),
            pltpu.VMEM((_LC,), f32),
            pltpu.VMEM((_LC,), f32),
            pltpu.VMEM((_LC,), f32),
            pltpu.VMEM((_LIST,), jnp.int32),
            pltpu.VMEM((16,), f32),
            pltpu.SemaphoreType.DMA,
        ],
    )(_sc_body)
    return fn(qr, t1i, t2i, t1v, t2v, qn2, qsum, colmax, kn2, ksum)


def kernel(query, keys_b):
    keys = keys_b[0]
    B, d, h, w = query.shape
    qt = jnp.transpose(query, (0, 2, 3, 1)).reshape(B * h * w, d)
    (p, concat, qr, t1i, t2i, t1v, t2v, qn2, qsum,
     colmax, kn2, ksum) = _tc_call(qt, keys)
    upd, lossp = _sc_call(
        qr,
        t1i.reshape(_N), t2i.reshape(_N),
        t1v.reshape(_N), t2v.reshape(_N),
        qn2.reshape(_N), qsum.reshape(_N),
        colmax.reshape(_M), kn2.reshape(_M), ksum.reshape(_M),
    )
    updated_memory_b = jnp.broadcast_to(upd[None], (B, _M, _D))
    score_memory_b = p.reshape(B, h, w, _M)
    updated_query = jnp.concatenate([qr, concat], axis=1).reshape(
        B, h, w, 2 * d).transpose(0, 3, 1, 2)
    loss_list = lossp[:, :2].reshape(B, 4, 2).sum(axis=1) / (h * w)
    return updated_query, updated_memory_b, score_memory_b, loss_list
